# Initial kernel scaffold; baseline (speedup 1.0000x reference)
import jax
import jax.numpy as jnp
from jax.experimental import pallas as pl


def kernel(vi, atom_mol_batch, N, W, b):
    raise NotImplementedError("write your pallas kernel here")


# speedup vs baseline: 1.6618x; 1.6618x over previous
import functools

import jax
import jax.numpy as jnp
from jax import lax
from jax.experimental import pallas as pl
from jax.experimental.pallas import tpu as pltpu
from jax.experimental.pallas import tpu_sc as plsc


_BM = 512


def _mm_body(x_ref, w_ref, b_ref, o_ref):
    y = jnp.dot(x_ref[...], w_ref[...], preferred_element_type=jnp.float32)
    y = jnp.maximum(y + b_ref[...], 0.0)
    d_half = y.shape[1] // 2
    lane = lax.broadcasted_iota(jnp.int32, (y.shape[0], d_half), 1)
    cnt_blk = jnp.where(lane == 0, 1.0, 0.0).astype(jnp.float32)
    o_ref[0] = jnp.concatenate([y[:, :d_half], cnt_blk], axis=1)
    o_ref[1] = jnp.concatenate([y[:, d_half:], cnt_blk], axis=1)


def _matmul_relu_split(x, w, b):
    n_atoms, d_in = x.shape
    d_out = w.shape[1]
    grid = (n_atoms // _BM,)
    return pl.pallas_call(
        _mm_body,
        grid=grid,
        in_specs=[
            pl.BlockSpec((_BM, d_in), lambda k: (k, 0)),
            pl.BlockSpec((d_in, d_out), lambda k: (0, 0)),
            pl.BlockSpec((1, d_out), lambda k: (0, 0)),
        ],
        out_specs=pl.BlockSpec((2, _BM, d_out), lambda k: (0, k, 0)),
        out_shape=jax.ShapeDtypeStruct((2, n_atoms, d_out), jnp.float32),
    )(x, w, b.reshape(1, d_out))


_G = 128
_GPC = 8
_CHUNK = _G * _GPC
_PIECE = 256
_NSUB = 16
_MPT = 640
_MW = 624


def _segment_mean_sc(p_planes, idx2d, n_mols, n_atoms):
    d = p_planes.shape[2]
    n_rows = idx2d.shape[0]
    rpt = 160
    last_full_chunks = (n_atoms - rpt * _G * (_NSUB - 1)) // _CHUNK
    tail_row0 = rpt * (_NSUB - 1) + last_full_chunks * _GPC
    tail_atoms = n_atoms - tail_row0 * _G
    acc_rows = _MPT * _NSUB
    assert n_rows >= tail_row0 + _GPC and acc_rows >= n_mols

    mesh = plsc.VectorSubcoreMesh(core_axis_name="c", subcore_axis_name="s")

    @functools.partial(
        pl.kernel,
        mesh=mesh,
        out_type=jax.ShapeDtypeStruct((2, n_mols, d), jnp.float32),
        scratch_types=[
            pltpu.VMEM((320, d), jnp.float32),
            pltpu.VMEM((_GPC, _G), jnp.int32),
            pltpu.VMEM_SHARED((acc_rows, d), jnp.float32),
        ],
    )
    def seg_mean(p_hbm, idx_hbm, out_hbm, rows_v, idx_v, sums_sh):
        c = lax.axis_index("c")
        s = lax.axis_index("s")

        zeros16 = jnp.zeros((16,), jnp.float32)

        def zero_body(m, carry):
            for j in range(d // 16):
                rows_v[m, pl.ds(16 * j, 16)] = zeros16
            return carry

        lax.fori_loop(0, 320, zero_body, 0)
        z0 = s * _MPT
        pltpu.sync_copy(rows_v, sums_sh.at[pl.ds(z0, 320)])
        pltpu.sync_copy(rows_v, sums_sh.at[pl.ds(z0 + 320, 320)])
        plsc.subcore_barrier()

        row_base = s * rpt
        n_chunks = jnp.where(s == _NSUB - 1, last_full_chunks, rpt // _GPC)

        def chunk_body(t, carry):
            r0 = row_base + t * _GPC
            pltpu.sync_copy(idx_hbm.at[pl.ds(r0, _GPC)], idx_v)
            for piece in range(_CHUNK // _PIECE):
                pltpu.sync_copy(
                    p_hbm.at[c].at[pl.ds(r0 * _G + piece * _PIECE, _PIECE)],
                    rows_v.at[pl.ds(0, _PIECE)])
                for j in range(_PIECE // _G):
                    jj = piece * (_PIECE // _G) + j
                    pltpu.sync_copy(rows_v.at[pl.ds(j * _G, _G)],
                                    sums_sh.at[idx_v.at[jj]], add=True)
            return carry

        lax.fori_loop(0, n_chunks, chunk_body, 0)

        @pl.when(s == _NSUB - 1)
        def _tail():
            pltpu.sync_copy(idx_hbm.at[pl.ds(tail_row0, _GPC)], idx_v)
            for piece in range(tail_atoms // _PIECE):
                pltpu.sync_copy(
                    p_hbm.at[c].at[pl.ds(tail_row0 * _G + piece * _PIECE,
                                         _PIECE)],
                    rows_v.at[pl.ds(0, _PIECE)])
                for j in range(_PIECE // _G):
                    jj = piece * (_PIECE // _G) + j
                    pltpu.sync_copy(rows_v.at[pl.ds(j * _G, _G)],
                                    sums_sh.at[idx_v.at[jj]], add=True)

        plsc.subcore_barrier()

        m0 = s * _MW
        d_half = d // 2

        def div_body(m, carry):
            cv = rows_v[m, pl.ds(d_half, 16)]
            r = (1.0 / jnp.maximum(cv, 1.0))[0]
            for j in range(d // 16):
                rows_v[m, pl.ds(16 * j, 16)] = rows_v[m, pl.ds(16 * j, 16)] * r
            return carry

        pltpu.sync_copy(sums_sh.at[pl.ds(m0, 320)], rows_v)
        lax.fori_loop(0, 320, div_body, 0)
        pltpu.sync_copy(rows_v, out_hbm.at[c].at[pl.ds(m0, 320)])

        pltpu.sync_copy(sums_sh.at[pl.ds(m0 + 320, 320)], rows_v)
        lax.fori_loop(0, 320, div_body, 0)
        pltpu.sync_copy(rows_v.at[pl.ds(0, _MW - 320)],
                        out_hbm.at[c].at[pl.ds(m0 + 320, _MW - 320)])

        @pl.when(s == _NSUB - 1)
        def _write_tail():
            pltpu.sync_copy(
                rows_v.at[pl.ds(_MW - 320, _MPT - _MW)],
                out_hbm.at[c].at[pl.ds(m0 + _MW, _MPT - _MW)])

    return seg_mean(p_planes, idx2d)


def kernel(vi, atom_mol_batch, N, W, b):
    n_mols = N.shape[0]
    n_atoms = vi.shape[0] - n_mols
    atom_embed = vi[:-n_mols, :]
    p_planes = _matmul_relu_split(atom_embed, W, b)
    n_rows = -(-n_atoms // _CHUNK) * _GPC
    idx_pad = jnp.full((n_rows * _G - n_atoms,), n_mols, dtype=jnp.int32)
    idx2d = jnp.concatenate([atom_mol_batch, idx_pad]).reshape(n_rows, _G)
    halves = _segment_mean_sc(p_planes, idx2d, n_mols, n_atoms)
    d_half = W.shape[1] // 2
    return jnp.concatenate(
        [halves[0, :, :d_half], halves[1, :, :d_half]], axis=1)


# speedup vs baseline: 2.0625x; 1.2411x over previous
import functools

import jax
import jax.numpy as jnp
from jax import lax
from jax.experimental import pallas as pl
from jax.experimental.pallas import tpu as pltpu
from jax.experimental.pallas import tpu_sc as plsc


_BM = 512


def _mm_body(x_ref, w_ref, b_ref, o_ref):
    y = jnp.dot(x_ref[...], w_ref[...], preferred_element_type=jnp.float32)
    o_ref[...] = jnp.maximum(y + b_ref[...], 0.0)


def _matmul_relu(x, w, b):
    n_atoms, d_in = x.shape
    d_out = w.shape[1]
    grid = (n_atoms // _BM,)
    return pl.pallas_call(
        _mm_body,
        grid=grid,
        in_specs=[
            pl.BlockSpec((_BM, d_in), lambda k: (k, 0)),
            pl.BlockSpec((d_in, d_out), lambda k: (0, 0)),
            pl.BlockSpec((1, d_out), lambda k: (0, 0)),
        ],
        out_specs=pl.BlockSpec((_BM, d_out), lambda k: (k, 0)),
        out_shape=jax.ShapeDtypeStruct((n_atoms, d_out), jnp.float32),
    )(x, w, b.reshape(1, d_out))


_G = 128
_GPC = 8
_CHUNK = _G * _GPC
_NSUB = 16
_ACC = 10240
_CROWS = _ACC // _G


def _segment_partials_sc(p, idx2d, n_rows_real):
    d = p.shape[1]
    n_rows = idx2d.shape[0]
    n_workers = 2 * _NSUB
    rpw = 80
    mpt = _ACC // _NSUB
    mesh = plsc.VectorSubcoreMesh(core_axis_name="c", subcore_axis_name="s")

    @functools.partial(
        pl.kernel,
        mesh=mesh,
        out_type=(
            jax.ShapeDtypeStruct((2, _ACC, d), jnp.float32),
            jax.ShapeDtypeStruct((2, 1, _ACC), jnp.float32),
        ),
        scratch_types=[
            pltpu.VMEM((_G, d), jnp.float32),
            pltpu.VMEM((_GPC, _G), jnp.int32),
            pltpu.VMEM((_G,), jnp.float32),
            pltpu.VMEM((mpt,), jnp.float32),
            pltpu.VMEM_SHARED((_ACC, d), jnp.float32),
            pltpu.VMEM_SHARED((_ACC,), jnp.float32),
        ],
    )
    def seg_part(p_hbm, idx_hbm, sums_out, cnts_out, rows_v, idx_v, ones_v,
                 zero_v, sums_sh, cnts_sh):
        c = lax.axis_index("c")
        s = lax.axis_index("s")
        w = c * _NSUB + s

        zeros16 = jnp.zeros((16,), jnp.float32)
        ones16 = jnp.ones((16,), jnp.float32)

        def zero_body(m, carry):
            for j in range(d // 16):
                rows_v[m, pl.ds(16 * j, 16)] = zeros16
            return carry

        lax.fori_loop(0, _G, zero_body, 0)
        for k in range(_G // 16):
            ones_v[pl.ds(16 * k, 16)] = ones16
        for k in range(mpt // 16):
            zero_v[pl.ds(16 * k, 16)] = zeros16

        z0 = s * mpt
        for k in range(mpt // _G):
            pltpu.sync_copy(rows_v, sums_sh.at[pl.ds(z0 + k * _G, _G)])
        pltpu.sync_copy(zero_v, cnts_sh.at[pl.ds(z0, mpt)])
        plsc.subcore_barrier()

        row_base = w * rpw
        n_chunks = jnp.where(w == n_workers - 1,
                             (n_rows - (n_workers - 1) * rpw) // _GPC,
                             rpw // _GPC)

        def chunk_body(t, carry):
            r0 = row_base + t * _GPC
            pltpu.sync_copy(idx_hbm.at[pl.ds(r0, _GPC)], idx_v)
            for j in range(_GPC):
                @pl.when(r0 + j < n_rows_real)
                def _piece():
                    pltpu.sync_copy(p_hbm.at[pl.ds((r0 + j) * _G, _G)],
                                    rows_v)
                    pltpu.sync_copy(rows_v, sums_sh.at[idx_v.at[j]],
                                    add=True)
                    pltpu.sync_copy(ones_v, cnts_sh.at[idx_v.at[j]],
                                    add=True)
            return carry

        lax.fori_loop(0, n_chunks, chunk_body, 0)
        plsc.subcore_barrier()

        pltpu.sync_copy(sums_sh.at[pl.ds(z0, mpt)],
                        sums_out.at[c].at[pl.ds(z0, mpt)])

        @pl.when(s == 0)
        def _dump_counts():
            pltpu.sync_copy(cnts_sh, cnts_out.at[c].at[0])

    return seg_part(p, idx2d)


_CM = 1024


def _combine_body(s_ref, c_ref, o_ref):
    cs = c_ref[0] + c_ref[1]
    r = 1.0 / jnp.maximum(cs, 1.0)
    o_ref[...] = (s_ref[0] + s_ref[1]) * r[:, None]


def _combine(sums, cnts_flat, n_mols, d):
    grid = (-(-n_mols // _CM),)
    return pl.pallas_call(
        _combine_body,
        grid=grid,
        in_specs=[
            pl.BlockSpec((2, _CM, d), lambda k: (0, k, 0)),
            pl.BlockSpec((2, _CM), lambda k: (0, k)),
        ],
        out_specs=pl.BlockSpec((_CM, d), lambda k: (k, 0)),
        out_shape=jax.ShapeDtypeStruct((n_mols, d), jnp.float32),
    )(sums, cnts_flat)


def kernel(vi, atom_mol_batch, N, W, b):
    n_mols = N.shape[0]
    n_atoms = vi.shape[0] - n_mols
    d = W.shape[1]
    atom_embed = vi[:-n_mols, :]
    p = _matmul_relu(atom_embed, W, b)
    n_rows_real = -(-n_atoms // _G)
    n_rows = -(-n_atoms // _CHUNK) * _GPC
    idx_pad = jnp.full((n_rows * _G - n_atoms,), n_mols, dtype=jnp.int32)
    idx2d = jnp.concatenate([atom_mol_batch, idx_pad]).reshape(n_rows, _G)
    sums, cnts = _segment_partials_sc(p, idx2d, n_rows_real)
    return _combine(sums, cnts.reshape(2, _ACC), n_mols, d)


# speedup vs baseline: 3.5042x; 1.6990x over previous
import functools

import jax
import jax.numpy as jnp
from jax import lax
from jax.experimental import pallas as pl
from jax.experimental.pallas import tpu as pltpu
from jax.experimental.pallas import tpu_sc as plsc


_BM = 1280


def _mm_body(x_ref, w_ref, b_ref, o_ref):
    y = jnp.dot(x_ref[...], w_ref[...], preferred_element_type=jnp.float32)
    o_ref[...] = jnp.maximum(y + b_ref[...], 0.0)


def _matmul_relu(x, n_atoms, w, b):
    d_in = x.shape[1]
    d_out = w.shape[1]
    grid = (n_atoms // _BM,)
    return pl.pallas_call(
        _mm_body,
        grid=grid,
        in_specs=[
            pl.BlockSpec((_BM, d_in), lambda k: (k, 0)),
            pl.BlockSpec((d_in, d_out), lambda k: (0, 0)),
            pl.BlockSpec((1, d_out), lambda k: (0, 0)),
        ],
        out_specs=pl.BlockSpec((_BM, d_out), lambda k: (k, 0)),
        out_shape=jax.ShapeDtypeStruct((n_atoms, d_out), jnp.float32),
    )(x, w, b.reshape(1, d_out))


_G = 128
_GPC = 8
_CHUNK = _G * _GPC
_NSUB = 16
_ACC = 10240
_CROWS = _ACC // _G


def _segment_partials_sc(p, idx2d, n_rows_real):
    d = p.shape[1]
    n_rows = idx2d.shape[0]
    n_workers = 2 * _NSUB
    rpw = 80
    mpt = _ACC // _NSUB
    mesh = plsc.VectorSubcoreMesh(core_axis_name="c", subcore_axis_name="s")

    @functools.partial(
        pl.kernel,
        mesh=mesh,
        out_type=(
            jax.ShapeDtypeStruct((2, _ACC, d), jnp.float32),
            jax.ShapeDtypeStruct((2, 1, _ACC), jnp.float32),
        ),
        scratch_types=[
            pltpu.VMEM((_G, d), jnp.float32),
            pltpu.VMEM((_GPC, _G), jnp.int32),
            pltpu.VMEM((_G,), jnp.float32),
            pltpu.VMEM((mpt,), jnp.float32),
            pltpu.VMEM_SHARED((_ACC, d), jnp.float32),
            pltpu.VMEM_SHARED((_ACC,), jnp.float32),
        ],
    )
    def seg_part(p_hbm, idx_hbm, sums_out, cnts_out, rows_v, idx_v, ones_v,
                 zero_v, sums_sh, cnts_sh):
        c = lax.axis_index("c")
        s = lax.axis_index("s")
        w = c * _NSUB + s

        zeros16 = jnp.zeros((16,), jnp.float32)
        ones16 = jnp.ones((16,), jnp.float32)

        def zero_body(m, carry):
            for j in range(d // 16):
                rows_v[m, pl.ds(16 * j, 16)] = zeros16
            return carry

        lax.fori_loop(0, _G, zero_body, 0)
        for k in range(_G // 16):
            ones_v[pl.ds(16 * k, 16)] = ones16
        for k in range(mpt // 16):
            zero_v[pl.ds(16 * k, 16)] = zeros16

        z0 = s * mpt
        for k in range(mpt // _G):
            pltpu.sync_copy(rows_v, sums_sh.at[pl.ds(z0 + k * _G, _G)])
        pltpu.sync_copy(zero_v, cnts_sh.at[pl.ds(z0, mpt)])
        plsc.subcore_barrier()

        row_base = w * rpw
        n_chunks = jnp.where(w == n_workers - 1,
                             (n_rows - (n_workers - 1) * rpw) // _GPC,
                             rpw // _GPC)

        def chunk_body(t, carry):
            r0 = row_base + t * _GPC
            pltpu.sync_copy(idx_hbm.at[pl.ds(r0, _GPC)], idx_v)
            for j in range(_GPC):
                @pl.when(r0 + j < n_rows_real)
                def _piece():
                    pltpu.sync_copy(p_hbm.at[pl.ds((r0 + j) * _G, _G)],
                                    rows_v)
                    pltpu.sync_copy(rows_v, sums_sh.at[idx_v.at[j]],
                                    add=True)
                    pltpu.sync_copy(ones_v, cnts_sh.at[idx_v.at[j]],
                                    add=True)
            return carry

        lax.fori_loop(0, n_chunks, chunk_body, 0)
        plsc.subcore_barrier()

        pltpu.sync_copy(sums_sh.at[pl.ds(z0, mpt)],
                        sums_out.at[c].at[pl.ds(z0, mpt)])

        @pl.when(s == 0)
        def _dump_counts():
            pltpu.sync_copy(cnts_sh, cnts_out.at[c].at[0])

    return seg_part(p, idx2d)


_CM = 1024


def _combine_body(s_ref, c_ref, o_ref):
    cs = c_ref[0] + c_ref[1]
    r = 1.0 / jnp.maximum(cs, 1.0)
    o_ref[...] = (s_ref[0] + s_ref[1]) * r[:, None]


def _combine(sums, cnts_flat, n_mols, d):
    grid = (-(-n_mols // _CM),)
    return pl.pallas_call(
        _combine_body,
        grid=grid,
        in_specs=[
            pl.BlockSpec((2, _CM, d), lambda k: (0, k, 0)),
            pl.BlockSpec((2, _CM), lambda k: (0, k)),
        ],
        out_specs=pl.BlockSpec((_CM, d), lambda k: (k, 0)),
        out_shape=jax.ShapeDtypeStruct((n_mols, d), jnp.float32),
    )(sums, cnts_flat)


def kernel(vi, atom_mol_batch, N, W, b):
    n_mols = N.shape[0]
    n_atoms = vi.shape[0] - n_mols
    d = W.shape[1]
    p = _matmul_relu(vi, n_atoms, W, b)
    n_rows_real = -(-n_atoms // _G)
    n_rows = -(-n_atoms // _CHUNK) * _GPC
    idx_pad = jnp.full((n_rows * _G - n_atoms,), n_mols, dtype=jnp.int32)
    idx2d = jnp.concatenate([atom_mol_batch, idx_pad]).reshape(n_rows, _G)
    sums, cnts = _segment_partials_sc(p, idx2d, n_rows_real)
    return _combine(sums, cnts.reshape(2, _ACC), n_mols, d)


# speedup vs baseline: 4.2360x; 1.2089x over previous
import functools

import jax
import jax.numpy as jnp
from jax import lax
from jax.experimental import pallas as pl
from jax.experimental.pallas import tpu as pltpu
from jax.experimental.pallas import tpu_sc as plsc

_K = 2
_BM = 1280
_G = 128
_GPC = 8
_CHUNK = _G * _GPC
_NSUB = 16
_ACC = 10240


def _mm_body(x_ref, w_ref, b_ref, o_ref):
    y = jnp.dot(x_ref[...], w_ref[...], preferred_element_type=jnp.float32)
    o_ref[...] = jnp.maximum(y + b_ref[...], 0.0)


def _matmul_relu(x, row0, rows, w, b):
    d_in = x.shape[1]
    d_out = w.shape[1]
    off = row0 // _BM
    return pl.pallas_call(
        _mm_body,
        grid=(rows // _BM,),
        in_specs=[
            pl.BlockSpec((_BM, d_in), lambda k: (k + off, 0)),
            pl.BlockSpec((d_in, d_out), lambda k: (0, 0)),
            pl.BlockSpec((1, d_out), lambda k: (0, 0)),
        ],
        out_specs=pl.BlockSpec((_BM, d_out), lambda k: (k, 0)),
        out_shape=jax.ShapeDtypeStruct((rows, d_out), jnp.float32),
    )(x, w, b.reshape(1, d_out))


def _segment_partials_sc(p, idx2d, n_rows_real):
    d = p.shape[1]
    n_rows = idx2d.shape[0]
    n_workers = 2 * _NSUB
    rpw = _GPC * (-(-n_rows_real // (_GPC * n_workers)))
    mpt = _ACC // _NSUB
    assert (n_workers - 1) * rpw <= n_rows
    mesh = plsc.VectorSubcoreMesh(core_axis_name="c", subcore_axis_name="s")

    @functools.partial(
        pl.kernel,
        mesh=mesh,
        out_type=(
            jax.ShapeDtypeStruct((2, _ACC, d), jnp.float32),
            jax.ShapeDtypeStruct((2, 1, _ACC), jnp.float32),
        ),
        scratch_types=[
            pltpu.VMEM((_G, d), jnp.float32),
            pltpu.VMEM((_GPC, _G), jnp.int32),
            pltpu.VMEM((_G,), jnp.float32),
            pltpu.VMEM((mpt,), jnp.float32),
            pltpu.VMEM_SHARED((_ACC, d), jnp.float32),
            pltpu.VMEM_SHARED((_ACC,), jnp.float32),
        ],
    )
    def seg_part(p_hbm, idx_hbm, sums_out, cnts_out, rows_v, idx_v, ones_v,
                 zero_v, sums_sh, cnts_sh):
        c = lax.axis_index("c")
        s = lax.axis_index("s")
        w = c * _NSUB + s

        zeros16 = jnp.zeros((16,), jnp.float32)
        ones16 = jnp.ones((16,), jnp.float32)

        def zero_body(m, carry):
            for j in range(d // 16):
                rows_v[m, pl.ds(16 * j, 16)] = zeros16
            return carry

        lax.fori_loop(0, _G, zero_body, 0)
        for k in range(_G // 16):
            ones_v[pl.ds(16 * k, 16)] = ones16
        for k in range(mpt // 16):
            zero_v[pl.ds(16 * k, 16)] = zeros16

        z0 = s * mpt
        for k in range(mpt // _G):
            pltpu.sync_copy(rows_v, sums_sh.at[pl.ds(z0 + k * _G, _G)])
        pltpu.sync_copy(zero_v, cnts_sh.at[pl.ds(z0, mpt)])
        plsc.subcore_barrier()

        row_base = w * rpw
        n_chunks = jnp.where(w == n_workers - 1,
                             (n_rows - (n_workers - 1) * rpw) // _GPC,
                             rpw // _GPC)

        def chunk_body(t, carry):
            r0 = row_base + t * _GPC
            pltpu.sync_copy(idx_hbm.at[pl.ds(r0, _GPC)], idx_v)
            for j in range(_GPC):
                @pl.when(r0 + j < n_rows_real)
                def _piece():
                    pltpu.sync_copy(p_hbm.at[pl.ds((r0 + j) * _G, _G)],
                                    rows_v)
                    pltpu.sync_copy(rows_v, sums_sh.at[idx_v.at[j]],
                                    add=True)
                    pltpu.sync_copy(ones_v, cnts_sh.at[idx_v.at[j]],
                                    add=True)
            return carry

        lax.fori_loop(0, n_chunks, chunk_body, 0)
        plsc.subcore_barrier()

        pltpu.sync_copy(sums_sh.at[pl.ds(z0, mpt)],
                        sums_out.at[c].at[pl.ds(z0, mpt)])

        @pl.when(s == 0)
        def _dump_counts():
            pltpu.sync_copy(cnts_sh, cnts_out.at[c].at[0])

    return seg_part(p, idx2d)


_CM = 1024


def _combine_body(*refs):
    s_refs = refs[:_K]
    c_refs = refs[_K:2 * _K]
    o_ref = refs[2 * _K]
    cs = sum(c_ref[0] + c_ref[1] for c_ref in c_refs)
    ss = sum(s_ref[0] + s_ref[1] for s_ref in s_refs)
    r = 1.0 / jnp.maximum(cs, 1.0)
    o_ref[...] = ss * r[:, None]


def _combine(sums_list, cnts_list, n_mols, d):
    grid = (-(-n_mols // _CM),)
    return pl.pallas_call(
        _combine_body,
        grid=grid,
        in_specs=(
            [pl.BlockSpec((2, _CM, d), lambda k: (0, k, 0))] * _K
            + [pl.BlockSpec((2, _CM), lambda k: (0, k))] * _K
        ),
        out_specs=pl.BlockSpec((_CM, d), lambda k: (k, 0)),
        out_shape=jax.ShapeDtypeStruct((n_mols, d), jnp.float32),
    )(*sums_list, *cnts_list)


def kernel(vi, atom_mol_batch, N, W, b):
    n_mols = N.shape[0]
    n_atoms = vi.shape[0] - n_mols
    d = W.shape[1]
    cr = n_atoms // _K
    sums_list, cnts_list = [], []
    for k in range(_K):
        p_k = _matmul_relu(vi, k * cr, cr, W, b)
        n_rows_real = -(-cr // _G)
        n_rows = -(-cr // _CHUNK) * _GPC
        idx_k = lax.dynamic_slice_in_dim(atom_mol_batch, k * cr, cr)
        idx_pad = jnp.full((n_rows * _G - cr,), n_mols, dtype=jnp.int32)
        idx2d = jnp.concatenate([idx_k, idx_pad]).reshape(n_rows, _G)
        s_k, c_k = _segment_partials_sc(p_k, idx2d, n_rows_real)
        sums_list.append(s_k)
        cnts_list.append(c_k.reshape(2, _ACC))
    return _combine(sums_list, cnts_list, n_mols, d)
